# baseline (speedup 1.0000x reference)
import functools

import jax
import jax.numpy as jnp
from jax import lax
from jax.experimental import pallas as pl
from jax.experimental.pallas import tpu as pltpu
from jax.experimental.pallas import tpu_sc as plsc

B, N, C = 8, 20000, 80
NC, NS = 2, 16
NW = NC * NS
CH = 80
L = 16


def _sc_gather(sel_t, boxes_flat, scores_flat):
    k = sel_t.shape[1]
    p = -(-k // NW) // 8 * 8 + 8
    t_sub = -(-p // CH)
    p = t_sub * CH
    assert (k - p) % 8 == 0 and NW * p >= k

    mesh = plsc.VectorSubcoreMesh(
        core_axis_name="c", subcore_axis_name="s",
        num_cores=NC, num_subcores=NS)

    @functools.partial(
        pl.kernel,
        mesh=mesh,
        compiler_params=pltpu.CompilerParams(use_tc_tiling_on_sc=False),
        out_type=(
            jax.ShapeDtypeStruct((k * 5,), jnp.float32),
            jax.ShapeDtypeStruct((k,), jnp.int32),
            jax.ShapeDtypeStruct((k,), jnp.int32),
        ),
        scratch_types=[
            pltpu.VMEM((p,), jnp.int32),
            pltpu.VMEM((p,), jnp.int32),
            pltpu.VMEM((p,), jnp.int32),
            pltpu.VMEM((4, t_sub, CH), jnp.int32),
            pltpu.VMEM((t_sub, CH), jnp.int32),
            pltpu.VMEM((5, t_sub, CH), jnp.int32),
            pltpu.VMEM((5, p), jnp.float32),
            pltpu.SemaphoreType.DMA,
        ],
    )
    def body(sel_h, boxes_h, scores_h, dets_h, binds_h, cinds_h,
             bcol_v, ccol_v, ncol_v, bidx_v, sidx_v, oidx_v, cols_v, sem):
        wid = lax.axis_index("s") * NC + lax.axis_index("c")
        base = jnp.minimum(wid * p, k - p)
        in_cp = [
            pltpu.async_copy(sel_h.at[0, pl.ds(base, p)], bcol_v, sem),
            pltpu.async_copy(sel_h.at[1, pl.ds(base, p)], ccol_v, sem),
            pltpu.async_copy(sel_h.at[2, pl.ds(base, p)], ncol_v, sem),
        ]
        for cp in in_cp:
            cp.wait()

        iota = lax.iota(jnp.int32, L)
        for t in range(t_sub):
            for j in range(CH // L):
                src = pl.ds(t * CH + j * L, L)
                b = bcol_v[src]
                c = ccol_v[src]
                n = ncol_v[src]
                bn4 = (b * N + n) * 4
                out5 = (base + t * CH + j * L + iota) * 5
                for col in range(4):
                    bidx_v[col, t, pl.ds(j * L, L)] = bn4 + col
                    oidx_v[col, t, pl.ds(j * L, L)] = out5 + col
                sidx_v[t, pl.ds(j * L, L)] = (b * C + c) * N + n
                oidx_v[4, t, pl.ds(j * L, L)] = out5 + 4

        g_cp = []
        for t in range(t_sub):
            for col in range(4):
                g_cp.append(pltpu.async_copy(
                    boxes_h.at[bidx_v.at[col, t]],
                    cols_v.at[col, pl.ds(t * CH, CH)], sem))
            g_cp.append(pltpu.async_copy(
                scores_h.at[sidx_v.at[t]],
                cols_v.at[4, pl.ds(t * CH, CH)], sem))
        for cp in g_cp:
            cp.wait()

        s_cp = []
        for t in range(t_sub):
            for col in range(5):
                s_cp.append(pltpu.async_copy(
                    cols_v.at[col, pl.ds(t * CH, CH)],
                    dets_h.at[oidx_v.at[col, t]], sem))
        s_cp.append(pltpu.async_copy(bcol_v, binds_h.at[pl.ds(base, p)], sem))
        s_cp.append(pltpu.async_copy(ccol_v, cinds_h.at[pl.ds(base, p)], sem))
        for cp in s_cp:
            cp.wait()

    return body(sel_t, boxes_flat, scores_flat)


def kernel(boxes, scores, selected_indices):
    k = selected_indices.shape[0]
    sel_t = selected_indices.astype(jnp.int32).T
    dets_flat, binds, cinds = _sc_gather(
        sel_t, boxes.reshape(-1), scores.reshape(-1))
    return (dets_flat.reshape(k, 5),
            binds.astype(selected_indices.dtype),
            cinds.astype(selected_indices.dtype))


# speedup vs baseline: 1.7059x; 1.7059x over previous
import functools

import jax
import jax.numpy as jnp
import numpy as np
from jax import lax
from jax.experimental import pallas as pl
from jax.experimental.pallas import tpu as pltpu
from jax.experimental.pallas import tpu_sc as plsc

B, N, C = 8, 20000, 80
NC, NS = 2, 16
NW = NC * NS
CH = 80
L = 16


def _worker_rows(k):
    p = -(-k // NW) // 8 * 8 + 8
    t_sub = -(-p // CH)
    return t_sub * CH, t_sub


def _sc_gather(sel_flat, boxes_flat, scores_flat, asm_idx, *, k):
    p, t_sub = _worker_rows(k)
    assert (k - p) % 8 == 0 and NW * p >= k

    mesh = plsc.VectorSubcoreMesh(
        core_axis_name="c", subcore_axis_name="s",
        num_cores=NC, num_subcores=NS)

    @functools.partial(
        pl.kernel,
        mesh=mesh,
        compiler_params=pltpu.CompilerParams(use_tc_tiling_on_sc=False),
        out_type=(
            jax.ShapeDtypeStruct((k * 5,), jnp.float32),
            jax.ShapeDtypeStruct((k,), jnp.int32),
            jax.ShapeDtypeStruct((k,), jnp.int32),
        ),
        scratch_types=[
            pltpu.VMEM((3, t_sub, CH), jnp.int32),
            pltpu.VMEM((5 * t_sub, CH), jnp.int32),
            pltpu.VMEM((p,), jnp.int32),
            pltpu.VMEM((p,), jnp.int32),
            pltpu.VMEM((p,), jnp.int32),
            pltpu.VMEM((4, t_sub, CH), jnp.int32),
            pltpu.VMEM((t_sub, CH), jnp.int32),
            pltpu.VMEM((5 * p,), jnp.float32),
            pltpu.VMEM((5 * p,), jnp.float32),
            pltpu.VMEM_SHARED((NS * 5 * p,), jnp.float32),
            pltpu.SemaphoreType.DMA,
            pltpu.SemaphoreType.DMA,
            pltpu.SemaphoreType.DMA,
            pltpu.SemaphoreType.DMA,
            pltpu.SemaphoreType.DMA,
        ],
    )
    def body(sel_h, boxes_h, scores_h, asmidx_h,
             dets_h, binds_h, cinds_h,
             splitidx_v, asmidx_v, bcol_v, ccol_v, ncol_v,
             bidx_v, sidx_v, cols_v, dets_v, dets_sh,
             sem_a, sem_sp, sem_g, sem_s, sem_o):
        cid = lax.axis_index("c")
        sid = lax.axis_index("s")
        wid = sid * NC + cid
        base = jnp.minimum(wid * p, k - p)

        asm_cp = pltpu.async_copy(asmidx_h.at[sid], asmidx_v, sem_a)

        iota = lax.iota(jnp.int32, L)
        for t in range(t_sub):
            for j in range(CH // L):
                pos3 = 3 * (base + t * CH + j * L + iota)
                for col in range(3):
                    splitidx_v[col, t, pl.ds(j * L, L)] = pos3 + col

        sp_cp = []
        cols3 = [bcol_v, ccol_v, ncol_v]
        for col in range(3):
            for t in range(t_sub):
                sp_cp.append(pltpu.async_copy(
                    sel_h.at[splitidx_v.at[col, t]],
                    cols3[col].at[pl.ds(t * CH, CH)], sem_sp))
        for cp in sp_cp:
            cp.wait()

        for t in range(t_sub):
            for j in range(CH // L):
                src = pl.ds(t * CH + j * L, L)
                b = bcol_v[src]
                c = ccol_v[src]
                n = ncol_v[src]
                bn4 = (b * N + n) * 4
                for col in range(4):
                    bidx_v[col, t, pl.ds(j * L, L)] = bn4 + col
                sidx_v[t, pl.ds(j * L, L)] = (b * C + c) * N + n

        g_cp = []
        for t in range(t_sub):
            for col in range(4):
                g_cp.append(pltpu.async_copy(
                    boxes_h.at[bidx_v.at[col, t]],
                    cols_v.at[pl.ds(col * p + t * CH, CH)], sem_g))
            g_cp.append(pltpu.async_copy(
                scores_h.at[sidx_v.at[t]],
                cols_v.at[pl.ds(4 * p + t * CH, CH)], sem_g))
        asm_cp.wait()
        for cp in g_cp:
            cp.wait()

        s_cp = []
        for m in range(5 * t_sub):
            s_cp.append(pltpu.async_copy(
                cols_v.at[pl.ds(m * CH, CH)],
                dets_sh.at[asmidx_v.at[m]], sem_s))
        for cp in s_cp:
            cp.wait()
        plsc.subcore_barrier()
        pltpu.sync_copy(dets_sh.at[pl.ds(sid * 5 * p, 5 * p)], dets_v)

        out_cp = [
            pltpu.async_copy(dets_v, dets_h.at[pl.ds(5 * base, 5 * p)], sem_o),
            pltpu.async_copy(bcol_v, binds_h.at[pl.ds(base, p)], sem_o),
            pltpu.async_copy(ccol_v, cinds_h.at[pl.ds(base, p)], sem_o),
        ]
        for cp in out_cp:
            cp.wait()

    return body(sel_flat, boxes_flat, scores_flat, asm_idx)


def _asm_table(k):
    p, t_sub = _worker_rows(k)
    q = np.arange(5 * p, dtype=np.int32)
    col = q // p
    i = q % p
    local = i * 5 + col
    table = (np.arange(NS, dtype=np.int32)[:, None] * (5 * p) + local[None, :])
    return table.reshape(NS, 5 * t_sub, CH)


def kernel(boxes, scores, selected_indices):
    k = selected_indices.shape[0]
    dets_flat, binds, cinds = _sc_gather(
        selected_indices.astype(jnp.int32).reshape(-1),
        boxes.reshape(-1), scores.reshape(-1),
        jnp.asarray(_asm_table(k)), k=k)
    return (dets_flat.reshape(k, 5),
            binds.astype(selected_indices.dtype),
            cinds.astype(selected_indices.dtype))
